# baseline (speedup 1.0000x reference)
import jax
import jax.numpy as jnp
from jax.experimental import pallas as pl
from jax.experimental.pallas import tpu as pltpu
from jax.experimental.pallas import tpu_sc as plsc

_GATHER_WINDOW = 128
_BM = 1024


def _sc_gather(table, idx2d, n_idx, d):
    mesh = plsc.VectorSubcoreMesh(core_axis_name="core", subcore_axis_name="subcore")

    @pl.kernel(
        out_type=jax.ShapeDtypeStruct((n_idx, d), table.dtype),
        mesh=mesh,
        compiler_params=pltpu.CompilerParams(use_tc_tiling_on_sc=False),
    )
    def k(tab_hbm, i_hbm, o_hbm):
        def body(i_vmem, o_vmem):
            pltpu.sync_copy(tab_hbm.at[i_vmem.at[0]], o_vmem)

        pltpu.emit_pipeline(
            body,
            grid=(n_idx // _GATHER_WINDOW,),
            in_specs=[pl.BlockSpec((1, _GATHER_WINDOW), index_map=lambda i: (i, 0))],
            out_specs=[pl.BlockSpec((_GATHER_WINDOW, d), index_map=lambda i: (i, 0))],
            core_axis_name=("core", "subcore"),
            dimension_semantics=(pltpu.PARALLEL,),
        )(i_hbm, o_hbm)

    return k(table, idx2d)


def _tc_mlp(x, W1, b1, W2, b2):
    B, K = x.shape
    H = W1.shape[1]
    C = W2.shape[1]

    def body(x_ref, W1_ref, b1_ref, W2_ref, b2_ref, o_ref):
        xb = x_ref[...].astype(jnp.bfloat16)
        h = jnp.dot(xb, W1_ref[...], preferred_element_type=jnp.float32)
        h = jnp.maximum(h + b1_ref[...], 0.0).astype(jnp.bfloat16)
        o_ref[...] = jnp.dot(h, W2_ref[...], preferred_element_type=jnp.float32) + b2_ref[...]

    return pl.pallas_call(
        body,
        grid=(B // _BM,),
        in_specs=[
            pl.BlockSpec((_BM, K), lambda i: (i, 0)),
            pl.BlockSpec((K, H), lambda i: (0, 0)),
            pl.BlockSpec((1, H), lambda i: (0, 0)),
            pl.BlockSpec((H, C), lambda i: (0, 0)),
            pl.BlockSpec((1, C), lambda i: (0, 0)),
        ],
        out_specs=pl.BlockSpec((_BM, C), lambda i: (i, 0)),
        out_shape=jax.ShapeDtypeStruct((B, C), jnp.float32),
    )(x, W1.astype(jnp.bfloat16), b1.reshape(1, -1), W2.astype(jnp.bfloat16),
      b2.reshape(1, -1))


def kernel(w, embeddings, W1, b1, W2, b2):
    B, F = w.shape
    V, E = embeddings.shape
    idx = w.reshape(B * F // _GATHER_WINDOW, _GATHER_WINDOW).astype(jnp.int32)
    x = _sc_gather(embeddings, idx, B * F, E)
    x = x.reshape(B, F * E)
    return _tc_mlp(x, W1, b1, W2, b2)


# speedup vs baseline: 1.0674x; 1.0674x over previous
import jax
import jax.numpy as jnp
from jax import lax
from jax.experimental import pallas as pl
from jax.experimental.pallas import tpu as pltpu
from jax.experimental.pallas import tpu_sc as plsc

_WROWS = 8
_SUBGATHER = 96
_NBUF = 4
_BM = 1024


def _sc_gather(table, w, B, F, E):
    n_idx_blk = _WROWS * F
    out_rows = B * F * E // 128
    blk_out = n_idx_blk * E // 128
    mesh = plsc.VectorSubcoreMesh(core_axis_name="core", subcore_axis_name="subcore")

    n_sub = n_idx_blk // _SUBGATHER
    n_steps = B // _WROWS
    info = plsc.get_sparse_core_info()
    nw = info.num_cores * info.num_subcores
    spt = n_steps // nw
    NB = _NBUF

    blk128 = n_idx_blk * E // 128

    @pl.kernel(
        out_type=jax.ShapeDtypeStruct((n_steps, n_idx_blk, E), table.dtype),
        mesh=mesh,
        scratch_types=[
            pltpu.VMEM((NB, _WROWS, F), jnp.int32),
            pltpu.VMEM((NB, n_idx_blk, E), jnp.float32),
            pltpu.SemaphoreType.DMA((NB,)),
            pltpu.SemaphoreType.DMA((NB,)),
            pltpu.SemaphoreType.DMA((NB,)),
        ],
        compiler_params=pltpu.CompilerParams(use_tc_tiling_on_sc=False),
    )
    def k(tab_hbm, w_hbm, o_hbm, idx_v, rows_v, isem, gsem, osem):
        wid = lax.axis_index("subcore") * info.num_cores + lax.axis_index("core")
        base = wid * spt

        def idx_cp(s, b):
            return pltpu.make_async_copy(
                w_hbm.at[pl.ds((base + s) * _WROWS, _WROWS), :], idx_v.at[b],
                isem.at[b])

        def gath(b, j):
            return pltpu.make_async_copy(
                tab_hbm.at[idx_v.at[b, j]],
                rows_v.at[b, pl.ds(j * F, F), :],
                gsem.at[b],
            )

        def out_cp(s, b):
            return pltpu.make_async_copy(rows_v.at[b], o_hbm.at[base + s], osem.at[b])

        for b in range(NB):
            idx_cp(b, b).start()

        @pl.loop(0, spt // NB)
        def _(oi):
            for b in range(NB):
                s = oi * NB + b
                bp = (b - 1) % NB
                idx_cp(s, b).wait()

                @pl.when(oi > 0)
                def _():
                    out_cp(s - NB, b).wait()

                for j in range(_WROWS):
                    gath(b, j).start()

                @pl.when(s > 0)
                def _():
                    for j in range(_WROWS):
                        gath(bp, j).wait()

                    @pl.when(s - 1 + NB < spt)
                    def _():
                        idx_cp(s - 1 + NB, bp).start()

                    out_cp(s - 1, bp).start()

        bl = (spt - 1) % NB
        for j in range(_WROWS):
            gath(bl, j).wait()
        out_cp(spt - 1, bl).start()
        for b in range(NB):
            out_cp(spt - NB + b, (spt - NB + b) % NB).wait()

    return k(table, w)


def _tc_mlp(x, W1, b1, W2, b2):
    B, K = x.shape
    H = W1.shape[1]
    C = W2.shape[1]

    def body(x_ref, W1_ref, b1_ref, W2_ref, b2_ref, o_ref):
        xb = x_ref[...].astype(jnp.bfloat16)
        h = jnp.dot(xb, W1_ref[...], preferred_element_type=jnp.float32)
        h = jnp.maximum(h + b1_ref[...], 0.0).astype(jnp.bfloat16)
        o_ref[...] = jnp.dot(h, W2_ref[...], preferred_element_type=jnp.float32) + b2_ref[...]

    return pl.pallas_call(
        body,
        grid=(B // _BM,),
        in_specs=[
            pl.BlockSpec((_BM, K), lambda i: (i, 0)),
            pl.BlockSpec((K, H), lambda i: (0, 0)),
            pl.BlockSpec((1, H), lambda i: (0, 0)),
            pl.BlockSpec((H, C), lambda i: (0, 0)),
            pl.BlockSpec((1, C), lambda i: (0, 0)),
        ],
        out_specs=pl.BlockSpec((_BM, C), lambda i: (i, 0)),
        out_shape=jax.ShapeDtypeStruct((B, C), jnp.float32),
    )(x, W1.astype(jnp.bfloat16), b1.reshape(1, -1), W2.astype(jnp.bfloat16),
      b2.reshape(1, -1))


def kernel(w, embeddings, W1, b1, W2, b2):
    B, F = w.shape
    V, E = embeddings.shape
    x = _sc_gather(embeddings, w.astype(jnp.int32), B, F, E)
    x = x.reshape(B, F * E)
    return _tc_mlp(x, W1, b1, W2, b2)
